# baseline (speedup 1.0000x reference)
import functools

import jax
import jax.numpy as jnp
from jax import lax
from jax.experimental import pallas as pl
from jax.experimental.pallas import tpu as pltpu
from jax.experimental.pallas import tpu_sc as plsc

_NC = 2
_NS = 16
_NW = _NC * _NS
_L = 16


def _build(T, B, D, V):
    DPT = D // _NW

    mesh = plsc.VectorSubcoreMesh(core_axis_name="c", subcore_axis_name="s")

    @functools.partial(
        pl.kernel,
        out_type=jax.ShapeDtypeStruct((T, D, B), jnp.float32),
        mesh=mesh,
        scratch_types=[
            pltpu.VMEM((V,), jnp.float32),
            pltpu.VMEM((2 * B,), jnp.int32),
            pltpu.VMEM((2 * B,), jnp.float32),
            pltpu.VMEM_SHARED((T * B,), jnp.int32),
            pltpu.SemaphoreType.DMA,
            pltpu.SemaphoreType.DMA,
            pltpu.SemaphoreType.DMA,
        ],
        compiler_params=pltpu.CompilerParams(needs_layout_passes=False),
    )
    def emb(idxt_hbm, wt_hbm, out_hbm, row_v, idx2_v, out2_v, idx_sh, isem, osem, wsem):
        wid = lax.axis_index("s") * _NC + lax.axis_index("c")
        sid = lax.axis_index("s")

        pltpu.async_copy(wt_hbm.at[wid], row_v, wsem)

        chunk = (T * B) // _NS
        pltpu.sync_copy(
            idxt_hbm.at[pl.ds(sid * chunk, chunk)],
            idx_sh.at[pl.ds(sid * chunk, chunk)],
        )
        plsc.subcore_barrier()

        def drain_idx():
            pltpu.make_async_copy(
                idx_sh.at[pl.ds(0, B)], idx2_v.at[pl.ds(0, B)], isem
            ).wait()

        def drain_out():
            pltpu.make_async_copy(out2_v.at[pl.ds(0, B)], out_hbm.at[0].at[0], osem).wait()

        for di in range(DPT):
            d = wid + di * _NW
            pltpu.async_copy(idx_sh.at[pl.ds(0, B)], idx2_v.at[pl.ds(0, B)], isem)
            pltpu.make_async_copy(wt_hbm.at[d], row_v, wsem).wait()

            def per_t(t, carry):
                iu, pu = carry
                ib = iu * B
                pb = pu * B

                @pl.when(t + 1 < T)
                def _():
                    pltpu.async_copy(
                        idx_sh.at[pl.ds((t + 1) * B, B)],
                        idx2_v.at[pl.ds((1 - iu) * B, B)],
                        isem,
                    )

                drain_idx()

                @pl.when(t >= 2)
                def _():
                    drain_out()

                @plsc.parallel_loop(0, B, step=_L, unroll=32)
                def _(i):
                    out2_v[pl.ds(pb + i, _L)] = plsc.load_gather(
                        row_v, [idx2_v[pl.ds(ib + i, _L)]]
                    )

                pltpu.async_copy(
                    out2_v.at[pl.ds(pb, B)], out_hbm.at[t].at[d], osem
                )
                return (1 - iu, 1 - pu)

            lax.fori_loop(0, T, per_t, (0, 0), unroll=5)
            if di + 1 < DPT:
                pltpu.async_copy(wt_hbm.at[wid + (di + 1) * _NW], row_v, wsem)
            drain_out()
            drain_out()

    return emb


def kernel(idx, W):
    B, T = idx.shape
    V, D = W.shape
    idxt = idx.T.astype(jnp.int32).reshape(-1)
    wt = W.T
    out3 = _build(T, B, D, V)(idxt, wt)
    return out3.transpose(2, 0, 1)


# speedup vs baseline: 1.0216x; 1.0216x over previous
import functools

import jax
import jax.numpy as jnp
from jax import lax
from jax.experimental import pallas as pl
from jax.experimental.pallas import tpu as pltpu
from jax.experimental.pallas import tpu_sc as plsc

_NC = 2
_NS = 16
_NW = _NC * _NS
_L = 16


def _build(T, B, D, V):
    DPT = D // _NW

    mesh = plsc.VectorSubcoreMesh(core_axis_name="c", subcore_axis_name="s")

    @functools.partial(
        pl.kernel,
        out_type=jax.ShapeDtypeStruct((T, D, B), jnp.float32),
        mesh=mesh,
        scratch_types=[
            pltpu.VMEM((V,), jnp.float32),
            pltpu.VMEM((2 * B,), jnp.int32),
            pltpu.VMEM((2 * B,), jnp.float32),
            pltpu.VMEM_SHARED((T * B,), jnp.int32),
            pltpu.SemaphoreType.DMA,
            pltpu.SemaphoreType.DMA,
            pltpu.SemaphoreType.DMA,
        ],
        compiler_params=pltpu.CompilerParams(needs_layout_passes=False),
    )
    def emb(idxt_hbm, wt_hbm, out_hbm, row_v, idx2_v, out2_v, idx_sh, isem, osem, wsem):
        wid = lax.axis_index("s") * _NC + lax.axis_index("c")
        sid = lax.axis_index("s")

        pltpu.async_copy(wt_hbm.at[wid], row_v, wsem)

        chunk = (T * B) // _NS
        pltpu.sync_copy(
            idxt_hbm.at[pl.ds(sid * chunk, chunk)],
            idx_sh.at[pl.ds(sid * chunk, chunk)],
        )
        plsc.subcore_barrier()

        def drain_idx():
            pltpu.make_async_copy(
                idx_sh.at[pl.ds(0, B)], idx2_v.at[pl.ds(0, B)], isem
            ).wait()

        def drain_out():
            pltpu.make_async_copy(out2_v.at[pl.ds(0, B)], out_hbm.at[0].at[0], osem).wait()

        for di in range(DPT):
            d = wid + di * _NW
            pltpu.async_copy(idx_sh.at[pl.ds(0, B)], idx2_v.at[pl.ds(0, B)], isem)
            pltpu.make_async_copy(wt_hbm.at[d], row_v, wsem).wait()

            def per_t(t, carry):
                iu, pu = carry
                ib = iu * B
                pb = pu * B

                @pl.when(t + 1 < T)
                def _():
                    pltpu.async_copy(
                        idx_sh.at[pl.ds((t + 1) * B, B)],
                        idx2_v.at[pl.ds((1 - iu) * B, B)],
                        isem,
                    )

                drain_idx()

                @pl.when(t >= 2)
                def _():
                    drain_out()

                @plsc.parallel_loop(0, B, step=_L, unroll=32)
                def _(i):
                    out2_v[pl.ds(pb + i, _L)] = plsc.load_gather(
                        row_v, [idx2_v[pl.ds(ib + i, _L)]]
                    )

                pltpu.async_copy(
                    out2_v.at[pl.ds(pb, B)], out_hbm.at[t].at[d], osem
                )
                return (1 - iu, 1 - pu)

            lax.fori_loop(0, T, per_t, (0, 0), unroll=2)
            if di + 1 < DPT:
                pltpu.async_copy(wt_hbm.at[wid + (di + 1) * _NW], row_v, wsem)
            drain_out()
            drain_out()

    return emb


def kernel(idx, W):
    B, T = idx.shape
    V, D = W.shape
    idxt = idx.T.astype(jnp.int32).reshape(-1)
    wt = W.T
    out3 = _build(T, B, D, V)(idxt, wt)
    return out3.transpose(2, 0, 1)


# speedup vs baseline: 1.0224x; 1.0008x over previous
import functools

import jax
import jax.numpy as jnp
from jax import lax
from jax.experimental import pallas as pl
from jax.experimental.pallas import tpu as pltpu
from jax.experimental.pallas import tpu_sc as plsc

_NC = 2
_NS = 16
_NW = _NC * _NS
_L = 16


def _build(T, B, D, V):
    DPT = D // _NW

    mesh = plsc.VectorSubcoreMesh(core_axis_name="c", subcore_axis_name="s")

    @functools.partial(
        pl.kernel,
        out_type=jax.ShapeDtypeStruct((T, D, B), jnp.float32),
        mesh=mesh,
        scratch_types=[
            pltpu.VMEM((V,), jnp.float32),
            pltpu.VMEM((2 * B,), jnp.int32),
            pltpu.VMEM((2 * B,), jnp.float32),
            pltpu.VMEM_SHARED((T * B,), jnp.int32),
            pltpu.SemaphoreType.DMA,
            pltpu.SemaphoreType.DMA,
            pltpu.SemaphoreType.DMA,
        ],
        compiler_params=pltpu.CompilerParams(needs_layout_passes=False),
    )
    def emb(idxt_hbm, wt_hbm, out_hbm, row_v, idx2_v, out2_v, idx_sh, isem, osem, wsem):
        wid = lax.axis_index("s") * _NC + lax.axis_index("c")
        sid = lax.axis_index("s")

        pltpu.async_copy(wt_hbm.at[wid], row_v, wsem)

        chunk = (T * B) // _NS
        pltpu.sync_copy(
            idxt_hbm.at[pl.ds(sid * chunk, chunk)],
            idx_sh.at[pl.ds(sid * chunk, chunk)],
        )
        plsc.subcore_barrier()

        def drain_idx():
            pltpu.make_async_copy(
                idx_sh.at[pl.ds(0, B)], idx2_v.at[pl.ds(0, B)], isem
            ).wait()

        def drain_out():
            pltpu.make_async_copy(out2_v.at[pl.ds(0, B)], out_hbm.at[0].at[0], osem).wait()

        for di in range(DPT):
            d = wid + di * _NW
            pltpu.async_copy(idx_sh.at[pl.ds(0, B)], idx2_v.at[pl.ds(0, B)], isem)
            pltpu.make_async_copy(wt_hbm.at[d], row_v, wsem).wait()

            def per_t(t, carry):
                iu, pu = carry
                ib = iu * B
                pb = pu * B

                @pl.when(t + 1 < T)
                def _():
                    pltpu.async_copy(
                        idx_sh.at[pl.ds((t + 1) * B, B)],
                        idx2_v.at[pl.ds((1 - iu) * B, B)],
                        isem,
                    )

                drain_idx()

                @pl.when(t >= 2)
                def _():
                    drain_out()

                @plsc.parallel_loop(0, B, step=_L, unroll=64)
                def _(i):
                    out2_v[pl.ds(pb + i, _L)] = plsc.load_gather(
                        row_v, [idx2_v[pl.ds(ib + i, _L)]]
                    )

                pltpu.async_copy(
                    out2_v.at[pl.ds(pb, B)], out_hbm.at[t].at[d], osem
                )
                return (1 - iu, 1 - pu)

            lax.fori_loop(0, T, per_t, (0, 0))
            if di + 1 < DPT:
                pltpu.async_copy(wt_hbm.at[wid + (di + 1) * _NW], row_v, wsem)
            drain_out()
            drain_out()

    return emb


def kernel(idx, W):
    B, T = idx.shape
    V, D = W.shape
    idxt = idx.T.astype(jnp.int32).reshape(-1)
    wt = W.T
    out3 = _build(T, B, D, V)(idxt, wt)
    return out3.transpose(2, 0, 1)


# speedup vs baseline: 1.0287x; 1.0061x over previous
import functools

import jax
import jax.numpy as jnp
from jax import lax
from jax.experimental import pallas as pl
from jax.experimental.pallas import tpu as pltpu
from jax.experimental.pallas import tpu_sc as plsc

_NC = 2
_NS = 16
_NW = _NC * _NS
_L = 16


def _build(T, B, D, V):
    DPT = D // _NW

    mesh = plsc.VectorSubcoreMesh(core_axis_name="c", subcore_axis_name="s")

    @functools.partial(
        pl.kernel,
        out_type=jax.ShapeDtypeStruct((T, D, B), jnp.float32),
        mesh=mesh,
        scratch_types=[
            pltpu.VMEM((V,), jnp.float32),
            pltpu.VMEM((2 * B,), jnp.int32),
            pltpu.VMEM((2 * B,), jnp.float32),
            pltpu.VMEM_SHARED((T * B,), jnp.int32),
            pltpu.SemaphoreType.DMA,
            pltpu.SemaphoreType.DMA,
            pltpu.SemaphoreType.DMA,
        ],
        compiler_params=pltpu.CompilerParams(needs_layout_passes=False),
    )
    def emb(idxt_hbm, wt_hbm, out_hbm, row_v, idx2_v, out2_v, idx_sh, isem, osem, wsem):
        wid = lax.axis_index("s") * _NC + lax.axis_index("c")
        sid = lax.axis_index("s")

        pltpu.async_copy(wt_hbm.at[wid], row_v, wsem)

        chunk = (T * B) // _NS
        pltpu.sync_copy(
            idxt_hbm.at[pl.ds(sid * chunk, chunk)],
            idx_sh.at[pl.ds(sid * chunk, chunk)],
        )
        plsc.subcore_barrier()

        def drain_idx():
            pltpu.make_async_copy(
                idx_sh.at[pl.ds(0, B)], idx2_v.at[pl.ds(0, B)], isem
            ).wait()

        def drain_out():
            pltpu.make_async_copy(out2_v.at[pl.ds(0, B)], out_hbm.at[0].at[0], osem).wait()

        for di in range(DPT):
            d = wid + di * _NW
            pltpu.async_copy(idx_sh.at[pl.ds(0, B)], idx2_v.at[pl.ds(0, B)], isem)
            pltpu.make_async_copy(wt_hbm.at[d], row_v, wsem).wait()

            def per_t(t, carry):
                iu, pu = carry
                ib = iu * B
                pb = pu * B

                @pl.when(t + 1 < T)
                def _():
                    pltpu.async_copy(
                        idx_sh.at[pl.ds((t + 1) * B, B)],
                        idx2_v.at[pl.ds((1 - iu) * B, B)],
                        isem,
                    )

                drain_idx()

                @pl.when(t >= 2)
                def _():
                    drain_out()

                @plsc.parallel_loop(0, B, step=_L, unroll=32)
                def _(i):
                    out2_v[pl.ds(pb + i, _L)] = plsc.load_gather(
                        row_v, [idx2_v[pl.ds(ib + i, _L)]]
                    )

                pltpu.async_copy(
                    out2_v.at[pl.ds(pb, B)], out_hbm.at[t].at[d], osem
                )
                return (1 - iu, 1 - pu)

            lax.fori_loop(0, T, per_t, (0, 0))
            if di + 1 < DPT:
                pltpu.async_copy(wt_hbm.at[wid + (di + 1) * _NW], row_v, wsem)
            drain_out()
            drain_out()

    return emb


def kernel(idx, W):
    B, T = idx.shape
    V, D = W.shape
    idxt = idx.T.astype(jnp.int32).reshape(-1)
    wt = W.T
    out3 = _build(T, B, D, V)(idxt, wt)
    return out3.transpose(2, 0, 1)
